# baseline (speedup 1.0000x reference)
import functools

import jax
import jax.numpy as jnp
from jax.experimental import pallas as pl
from jax.experimental.pallas import tpu as pltpu

_WPAD = 40
_MT = 768


def _store_ranges(h, w, mt, n_m):
    ranges = []
    for m in range(n_m):
        lo, hi = m * mt, (m + 1) * mt
        rr = []
        for y in range(h):
            a, b = y * _WPAD, y * _WPAD + w
            s, e = max(a, lo), min(b, hi)
            if s < e:
                rr.append((s - lo, e - lo, y, s - a))
        ranges.append(rr)
    return ranges


def _fill_slab(slab_slot, src_ref, h, w):
    slab_slot[...] = jnp.zeros_like(slab_slot)
    for y in range(h):
        slab_slot[pl.ds((y + 1) * _WPAD + 1, w), :] = (
            src_ref[0, pl.ds(y * w, w), :])


def _rpn_body(x_ref, xn_ref, w1_ref, b1_ref, w2_ref, b2_ref, o_ref, xs_pad, *,
              h, w, n_m, n_b, ranges):
    c = w1_ref.shape[1]
    b = pl.program_id(0)
    cur = jax.lax.rem(b, 2)

    @pl.when(b == 0)
    def _build_first():
        _fill_slab(xs_pad.at[0], x_ref, h, w)

    @pl.when(b < n_b - 1)
    def _build_next():
        _fill_slab(xs_pad.at[1 - cur], xn_ref, h, w)

    for m in range(n_m):
        r0 = m * _MT
        acc = jnp.zeros((_MT, w1_ref.shape[2]), jnp.float32)
        for ky in range(3):
            xw = xs_pad[cur, pl.ds(r0 + ky * _WPAD, _MT + 8), :]
            for kx in range(3):
                xs = jax.lax.slice(xw, (kx, 0), (kx + _MT, c))
                acc = acc + jnp.dot(xs, w1_ref[3 * ky + kx],
                                    preferred_element_type=jnp.float32)
        hact = jnp.maximum(acc + b1_ref[0].astype(jnp.float32), 0.0)
        hact = hact.astype(jnp.bfloat16)
        outm = jnp.dot(hact, w2_ref[...], preferred_element_type=jnp.float32)
        outm = outm + b2_ref[0].astype(jnp.float32)
        for (sl, sh, y, xoff) in ranges[m]:
            o_ref[0, y, pl.ds(xoff, sh - sl), :] = (
                jax.lax.slice(outm, (sl, 0), (sh, outm.shape[1])))


def kernel(feats, W1, b1, W2, b2):
    B, C, H, W = feats.shape
    dim = W1.shape[0]
    co = W2.shape[0]

    nM = -(-(H * _WPAD) // _MT)
    Hpad = -(-(nM * _MT + 2 * _WPAD + 8) // _WPAD)
    Rpad = Hpad * _WPAD

    xc = jnp.transpose(feats.astype(jnp.bfloat16).reshape(B, C, H * W),
                       (0, 2, 1))
    w1 = jnp.transpose(W1, (2, 3, 1, 0)).reshape(9, C, dim)
    w1 = w1.astype(jnp.bfloat16)
    w2 = W2[:, :, 0, 0].T.astype(jnp.bfloat16)
    b1r = b1.reshape(1, dim)
    b2r = b2.reshape(1, co)

    body = functools.partial(_rpn_body, h=H, w=W, n_m=nM, n_b=B,
                             ranges=_store_ranges(H, W, _MT, nM))
    out = pl.pallas_call(
        body,
        grid=(B,),
        in_specs=[
            pl.BlockSpec((1, H * W, C), lambda b: (b, 0, 0)),
            pl.BlockSpec((1, H * W, C),
                         lambda b: (jnp.minimum(b + 1, B - 1), 0, 0)),
            pl.BlockSpec((9, C, dim), lambda b: (0, 0, 0)),
            pl.BlockSpec((1, dim), lambda b: (0, 0)),
            pl.BlockSpec((dim, co), lambda b: (0, 0)),
            pl.BlockSpec((1, co), lambda b: (0, 0)),
        ],
        out_specs=pl.BlockSpec((1, H, W, co), lambda b: (b, 0, 0, 0)),
        out_shape=jax.ShapeDtypeStruct((B, H, W, co), jnp.float32),
        scratch_shapes=[pltpu.VMEM((2, Rpad, C), jnp.bfloat16)],
    )(xc, xc, w1, b1r, w2, b2r)

    return out.reshape(B, H, W, co // 6, 6)


# speedup vs baseline: 1.0271x; 1.0271x over previous
import functools

import jax
import jax.numpy as jnp
from jax.experimental import pallas as pl
from jax.experimental.pallas import tpu as pltpu

_WPAD = 40
_MT = 512


def _store_ranges(h, w, mt, n_m):
    ranges = []
    for m in range(n_m):
        lo, hi = m * mt, (m + 1) * mt
        rr = []
        for y in range(h):
            a, b = y * _WPAD, y * _WPAD + w
            s, e = max(a, lo), min(b, hi)
            if s < e:
                rr.append((s - lo, e - lo, y, s - a))
        ranges.append(rr)
    return ranges


def _rpn_body(x_ref, w1_ref, b1_ref, w2_ref, b2_ref, o_ref, xs_pad, *,
              h, w, n_m, ranges):
    c = w1_ref.shape[1]

    xs_pad[...] = jnp.zeros_like(xs_pad)
    for y in range(h):
        xs_pad[pl.ds((y + 1) * _WPAD + 1, w), :] = x_ref[0, pl.ds(y * w, w), :]

    for m in range(n_m):
        r0 = m * _MT
        acc = jnp.zeros((_MT, w1_ref.shape[2]), jnp.float32)
        for ky in range(3):
            xw = xs_pad[pl.ds(r0 + ky * _WPAD, _MT + 8), :]
            for kx in range(3):
                xs = jax.lax.slice(xw, (kx, 0), (kx + _MT, c))
                acc = acc + jnp.dot(xs, w1_ref[3 * ky + kx],
                                    preferred_element_type=jnp.float32)
        hact = jnp.maximum(acc + b1_ref[0].astype(jnp.float32), 0.0)
        hact = hact.astype(jnp.bfloat16)
        outm = jnp.dot(hact, w2_ref[...], preferred_element_type=jnp.float32)
        outm = outm + b2_ref[0].astype(jnp.float32)
        for (sl, sh, y, xoff) in ranges[m]:
            o_ref[0, y, pl.ds(xoff, sh - sl), :] = (
                jax.lax.slice(outm, (sl, 0), (sh, outm.shape[1])))


def kernel(feats, W1, b1, W2, b2):
    B, C, H, W = feats.shape
    dim = W1.shape[0]
    co = W2.shape[0]

    nM = -(-(H * _WPAD) // _MT)
    Hpad = -(-(nM * _MT + 2 * _WPAD + 8) // _WPAD)
    Rpad = Hpad * _WPAD

    xc = jnp.transpose(feats.astype(jnp.bfloat16).reshape(B, C, H * W),
                       (0, 2, 1))
    w1 = jnp.transpose(W1, (2, 3, 1, 0)).reshape(9, C, dim)
    w1 = w1.astype(jnp.bfloat16)
    w2 = W2[:, :, 0, 0].T.astype(jnp.bfloat16)
    b1r = b1.reshape(1, dim)
    b2r = b2.reshape(1, co)

    body = functools.partial(_rpn_body, h=H, w=W, n_m=nM,
                             ranges=_store_ranges(H, W, _MT, nM))
    out = pl.pallas_call(
        body,
        grid=(B,),
        in_specs=[
            pl.BlockSpec((1, H * W, C), lambda b: (b, 0, 0)),
            pl.BlockSpec((9, C, dim), lambda b: (0, 0, 0)),
            pl.BlockSpec((1, dim), lambda b: (0, 0)),
            pl.BlockSpec((dim, co), lambda b: (0, 0)),
            pl.BlockSpec((1, co), lambda b: (0, 0)),
        ],
        out_specs=pl.BlockSpec((1, H, W, co), lambda b: (b, 0, 0, 0)),
        out_shape=jax.ShapeDtypeStruct((B, H, W, co), jnp.float32),
        scratch_shapes=[pltpu.VMEM((Rpad, C), jnp.bfloat16)],
    )(xc, w1, b1r, w2, b2r)

    return out.reshape(B, H, W, co // 6, 6)
